# baseline (speedup 1.0000x reference)
import functools

import jax
import jax.numpy as jnp
from jax import lax
from jax.experimental import pallas as pl
from jax.experimental.pallas import tpu as pltpu
from jax.experimental.pallas import tpu_sc as plsc

_NC = 2
_NS = 16
_NW = _NC * _NS
_L = 16

_ROW = 128
_RPC = 16
_SEG = _ROW // (2 * _L)


def _softplus_neg_abs(x):
    u = jnp.exp(-jnp.abs(x))
    t = u / (u + 2.0)
    t2 = t * t
    p = 1.0 + t2 * (1.0 / 3.0 + t2 * (0.2 + t2 * (1.0 / 7.0 + t2 * (1.0 / 9.0))))
    return 2.0 * t * p


def _bce_pair(a, b):
    pa = a > 0.0
    pb = b > 0.0
    na = a < 0.0
    nb = b < 0.0
    m = jnp.where((pa != pb) | (na != nb), 1.0, 0.0)
    t0 = jnp.where(pb, 1.0, 0.0)
    t1 = jnp.where(pa, 1.0, 0.0)
    bce = (jnp.maximum(a, 0.0) - a * t0 + _softplus_neg_abs(a)
           + jnp.maximum(b, 0.0) - b * t1 + _softplus_neg_abs(b))
    return bce, m


def _acc_segments(val, n_vals, carry):
    even = jnp.arange(0, 2 * _L, 2, dtype=jnp.int32)

    def seg(i, c):
        al, ac = c
        base = i * (2 * _L)
        a = plsc.load_gather(val, [base + even])
        b = plsc.load_gather(val, [base + even + 1])
        bce, m = _bce_pair(a, b)
        return (al + bce * m, ac + m)

    return lax.fori_loop(0, n_vals // (2 * _L), seg, carry)


def _sc_body(rows_total, sdf_hbm, edges_hbm, out_hbm, idx, val, out_v, sem):
    wid = lax.axis_index("s") * _NC + lax.axis_index("c")
    total_chunks = rows_total // _RPC
    tail_rows = rows_total % _RPC
    base_c = total_chunks // _NW
    rem_c = total_chunks % _NW
    my_chunks = base_c + jnp.where(wid < rem_c, 1, 0)
    c0 = wid * base_c + jnp.minimum(wid, rem_c)

    def chunk(g, carry):
        r0 = (c0 + g) * _RPC
        pltpu.sync_copy(edges_hbm.at[pl.ds(r0, _RPC)], idx)
        cps = []
        for r in range(_RPC):
            cps.append(pltpu.async_copy(
                sdf_hbm.at[idx.at[r]], val.at[pl.ds(r * _ROW, _ROW)], sem))
        for cp in cps:
            cp.wait()
        return _acc_segments(val, _RPC * _ROW, carry)

    def tail(_, carry):
        rt = total_chunks * _RPC
        pltpu.sync_copy(edges_hbm.at[pl.ds(rt, tail_rows)],
                        idx.at[pl.ds(0, tail_rows)])
        cps = []
        for r in range(tail_rows):
            cps.append(pltpu.async_copy(
                sdf_hbm.at[idx.at[r]], val.at[pl.ds(r * _ROW, _ROW)], sem))
        for cp in cps:
            cp.wait()
        return _acc_segments(val, tail_rows * _ROW, carry)

    zeros = jnp.zeros((_L,), jnp.float32)
    acc = lax.fori_loop(0, my_chunks, chunk, (zeros, zeros))
    if tail_rows:
        acc = lax.fori_loop(0, jnp.where(wid == 0, 1, 0), tail, acc)
    acc_l, acc_c = acc
    out_v[0, :] = acc_l
    out_v[1, :] = acc_c
    pltpu.sync_copy(out_v, out_hbm.at[wid])


def kernel(sdf, all_edges):
    n_vals = all_edges.shape[0] * all_edges.shape[1]
    assert n_vals % _ROW == 0
    rows_total = n_vals // _ROW
    edges = all_edges.reshape(rows_total, _ROW)

    mesh = plsc.VectorSubcoreMesh(core_axis_name="c", subcore_axis_name="s")
    run = pl.kernel(
        functools.partial(_sc_body, rows_total),
        out_type=jax.ShapeDtypeStruct((_NW, 2, _L), jnp.float32),
        mesh=mesh,
        scratch_types=[
            pltpu.VMEM((_RPC, _ROW), jnp.int32),
            pltpu.VMEM((_RPC * _ROW,), jnp.float32),
            pltpu.VMEM((2, _L), jnp.float32),
            pltpu.SemaphoreType.DMA,
        ],
        compiler_params=pltpu.CompilerParams(needs_layout_passes=False),
    )
    parts = run(sdf, edges)
    loss = jnp.sum(parts[:, 0, :])
    cnt = jnp.sum(parts[:, 1, :])
    return loss / jnp.maximum(cnt, 1.0)
